# Initial kernel scaffold; baseline (speedup 1.0000x reference)
import jax
import jax.numpy as jnp
from jax.experimental import pallas as pl


def kernel(x, edge_index, batch, W1, b1, W2, b2, W3, b3, Wm1, bm1, Wm2, bm2):
    raise NotImplementedError("write your pallas kernel here")


# speedup vs baseline: 40.0646x; 40.0646x over previous
import functools

import jax
import jax.numpy as jnp
from jax import lax
from jax.experimental import pallas as pl
from jax.experimental.pallas import tpu as pltpu
from jax.experimental.pallas import tpu_sc as plsc

N = 10000
E = 320000
G = 64
D = 128
H = 16
NC = 2
NS = 16
NW = NC * NS
EPW = E // NW
CH = 80
NCHUNK = EPW // CH
RPT = N // NS
DEG_PAD = 10240
DPT = DEG_PAD // NS

_mesh = plsc.VectorSubcoreMesh(core_axis_name="c", subcore_axis_name="s")
_sc_params = pltpu.CompilerParams(use_tc_tiling_on_sc=False)


@functools.partial(
    pl.kernel,
    out_type=jax.ShapeDtypeStruct((NC, NS, DPT), jnp.float32),
    mesh=_mesh,
    compiler_params=_sc_params,
    scratch_types=[
        pltpu.VMEM((NCHUNK, CH), jnp.int32),
        pltpu.VMEM((CH,), jnp.float32),
        pltpu.VMEM((DPT,), jnp.float32),
        pltpu.VMEM_SHARED((DEG_PAD,), jnp.float32),
    ],
)
def _deg_kernel(dst_hbm, out_hbm, dst_v, ones_v, stage_v, acc_sh):
    c = lax.axis_index("c")
    s = lax.axis_index("s")

    def fill(i, _):
        ones_v[pl.ds(i * 16, 16)] = jnp.ones((16,), jnp.float32)
        return 0

    lax.fori_loop(0, CH // 16, fill, 0)

    def zero(i, _):
        stage_v[pl.ds(i * 16, 16)] = jnp.zeros((16,), jnp.float32)
        return 0

    lax.fori_loop(0, DPT // 16, zero, 0)
    pltpu.sync_copy(stage_v, acc_sh.at[pl.ds(s * DPT, DPT)])
    plsc.subcore_barrier()

    wid = s * NC + c
    pltpu.sync_copy(dst_hbm.at[wid], dst_v)

    def body(j, _):
        pltpu.sync_copy(ones_v, acc_sh.at[dst_v.at[j]], add=True)
        return 0

    lax.fori_loop(0, NCHUNK, body, 0)
    plsc.subcore_barrier()
    pltpu.sync_copy(acc_sh.at[pl.ds(s * DPT, DPT)], stage_v)
    pltpu.sync_copy(stage_v, out_hbm.at[c, s])


@functools.partial(
    pl.kernel,
    out_type=jax.ShapeDtypeStruct((NC, NS, RPT, H), jnp.float32),
    mesh=_mesh,
    compiler_params=_sc_params,
    scratch_types=[
        pltpu.VMEM((NCHUNK, CH), jnp.int32),
        pltpu.VMEM((NCHUNK, CH), jnp.int32),
        pltpu.VMEM((CH, H), jnp.float32),
        pltpu.VMEM((CH, H), jnp.float32),
        pltpu.VMEM((RPT, H), jnp.float32),
        pltpu.VMEM_SHARED((N, H), jnp.float32),
        pltpu.SemaphoreType.DMA,
        pltpu.SemaphoreType.DMA,
    ],
)
def _edge_kernel(y_hbm, src_hbm, dst_hbm, out_hbm,
                 src_v, dst_v, rows0, rows1, stage, acc_sh, semA, semB):
    c = lax.axis_index("c")
    s = lax.axis_index("s")

    def zero(i, _):
        stage[i, :] = jnp.zeros((H,), jnp.float32)
        return 0

    lax.fori_loop(0, RPT, zero, 0)
    pltpu.sync_copy(stage, acc_sh.at[pl.ds(s * RPT, RPT)])
    plsc.subcore_barrier()

    wid = s * NC + c
    pltpu.sync_copy(src_hbm.at[wid], src_v)
    pltpu.sync_copy(dst_hbm.at[wid], dst_v)

    pltpu.async_copy(y_hbm.at[src_v.at[0]], rows0, semA)

    def body(i, _):
        j = i * 2
        pltpu.async_copy(y_hbm.at[src_v.at[j + 1]], rows1, semB)
        pltpu.make_async_copy(y_hbm.at[src_v.at[0]], rows0, semA).wait()
        pltpu.sync_copy(rows0, acc_sh.at[dst_v.at[j]], add=True)
        pltpu.async_copy(y_hbm.at[src_v.at[j + 2]], rows0, semA)
        pltpu.make_async_copy(y_hbm.at[src_v.at[0]], rows1, semB).wait()
        pltpu.sync_copy(rows1, acc_sh.at[dst_v.at[j + 1]], add=True)
        return 0

    lax.fori_loop(0, NCHUNK // 2, body, 0)
    pltpu.make_async_copy(y_hbm.at[src_v.at[0]], rows0, semA).wait()
    pltpu.sync_copy(rows0, acc_sh.at[dst_v.at[NCHUNK - 1]], add=True)

    plsc.subcore_barrier()
    pltpu.sync_copy(acc_sh.at[pl.ds(s * RPT, RPT)], stage)
    pltpu.sync_copy(stage, out_hbm.at[c, s])


def _prep_body(cnt_ref, x_ref, w1_ref, y1_ref, dis_ref):
    cnt = cnt_ref[0] + cnt_ref[1]
    deg = cnt[:N] + 1.0
    dis = lax.rsqrt(deg)
    xw = jnp.dot(x_ref[...], w1_ref[...], preferred_element_type=jnp.float32)
    y1_ref[...] = xw * dis[:, None]
    dis_ref[...] = dis


def _mid_body(acc_ref, y_ref, dis_ref, b_ref, w_ref, o_ref):
    dis = dis_ref[...][:, None]
    pre = (acc_ref[0] + acc_ref[1] + y_ref[...]) * dis + b_ref[...]
    h = jnp.maximum(pre, 0.0)
    o_ref[...] = jnp.dot(h, w_ref[...],
                         preferred_element_type=jnp.float32) * dis


def _head_body(acc_ref, y_ref, dis_ref, b3_ref, batch_ref,
               wm1_ref, bm1_ref, wm2_ref, bm2_ref, o_ref):
    dis = dis_ref[...][:, None]
    h3 = (acc_ref[0] + acc_ref[1] + y_ref[...]) * dis + b3_ref[...]
    gid = lax.broadcasted_iota(jnp.int32, (G, N), 0)
    onehot = (gid == batch_ref[...][None, :]).astype(jnp.float32)
    sums = jnp.dot(onehot, h3, preferred_element_type=jnp.float32)
    counts = jnp.sum(onehot, axis=1)
    pooled = sums / jnp.maximum(counts, 1.0)[:, None]
    z = jnp.maximum(
        jnp.dot(pooled, wm1_ref[...], preferred_element_type=jnp.float32)
        + bm1_ref[...], 0.0)
    z = jnp.dot(z, wm2_ref[...],
                preferred_element_type=jnp.float32) + bm2_ref[...]
    m = jnp.max(z, axis=-1, keepdims=True)
    e = z - m
    o_ref[...] = e - jnp.log(jnp.sum(jnp.exp(e), axis=-1, keepdims=True))


_prep = pl.pallas_call(
    _prep_body,
    out_shape=(jax.ShapeDtypeStruct((N, H), jnp.float32),
               jax.ShapeDtypeStruct((N,), jnp.float32)),
)

_mid = pl.pallas_call(
    _mid_body,
    out_shape=jax.ShapeDtypeStruct((N, H), jnp.float32),
)

_head = pl.pallas_call(
    _head_body,
    out_shape=jax.ShapeDtypeStruct((G, 10), jnp.float32),
)


def kernel(x, edge_index, batch, W1, b1, W2, b2, W3, b3, Wm1, bm1, Wm2, bm2):
    src3d = edge_index[0].astype(jnp.int32).reshape(NW, NCHUNK, CH)
    dst3d = edge_index[1].astype(jnp.int32).reshape(NW, NCHUNK, CH)
    cnt = _deg_kernel(dst3d).reshape(NC, DEG_PAD)
    y1, dis = _prep(cnt, x, W1)
    acc1 = _edge_kernel(y1, src3d, dst3d).reshape(NC, N, H)
    y2 = _mid(acc1, y1, dis, b1, W2)
    acc2 = _edge_kernel(y2, src3d, dst3d).reshape(NC, N, H)
    y3 = _mid(acc2, y2, dis, b2, W3)
    acc3 = _edge_kernel(y3, src3d, dst3d).reshape(NC, N, H)
    return _head(acc3, y3, dis, b3, batch.astype(jnp.int32), Wm1, bm1, Wm2, bm2)


# speedup vs baseline: 64.2049x; 1.6025x over previous
import functools

import jax
import jax.numpy as jnp
from jax import lax
from jax.experimental import pallas as pl
from jax.experimental.pallas import tpu as pltpu
from jax.experimental.pallas import tpu_sc as plsc

N = 10000
E = 320000
G = 64
D = 128
H = 16
NC = 2
NS = 16
NW = NC * NS
EPW = E // NW
CH = 80
NCHUNK = EPW // CH
RPT = N // NS
DEG_PAD = 10240
DPT = DEG_PAD // NS

_mesh = plsc.VectorSubcoreMesh(core_axis_name="c", subcore_axis_name="s")
_sc_params = pltpu.CompilerParams(use_tc_tiling_on_sc=False)


@functools.partial(
    pl.kernel,
    out_type=jax.ShapeDtypeStruct((NC, NS, DPT), jnp.float32),
    mesh=_mesh,
    compiler_params=_sc_params,
    scratch_types=[
        pltpu.VMEM((NCHUNK, CH), jnp.int32),
        pltpu.VMEM((CH,), jnp.float32),
        pltpu.VMEM((DPT,), jnp.float32),
        pltpu.VMEM_SHARED((DEG_PAD,), jnp.float32),
    ] + [pltpu.SemaphoreType.DMA] * 8,
)
def _deg_kernel(dst_hbm, out_hbm, dst_v, ones_v, stage_v, acc_sh, *ssem):
    c = lax.axis_index("c")
    s = lax.axis_index("s")

    def fill(i, _):
        ones_v[pl.ds(i * 16, 16)] = jnp.ones((16,), jnp.float32)
        return 0

    lax.fori_loop(0, CH // 16, fill, 0)

    def zero(i, _):
        stage_v[pl.ds(i * 16, 16)] = jnp.zeros((16,), jnp.float32)
        return 0

    lax.fori_loop(0, DPT // 16, zero, 0)
    pltpu.sync_copy(stage_v, acc_sh.at[pl.ds(s * DPT, DPT)])
    plsc.subcore_barrier()

    wid = s * NC + c
    pltpu.sync_copy(dst_hbm.at[wid], dst_v)

    def body(g, _):
        for b in range(8):
            j = g * 8 + b

            @pl.when(j < NCHUNK)
            def _():
                @pl.when(j >= 8)
                def _():
                    pltpu.make_async_copy(
                        ones_v, acc_sh.at[dst_v.at[0]], ssem[b]).wait()

                pltpu.async_copy(
                    ones_v, acc_sh.at[dst_v.at[j]], ssem[b], add=True)

        return 0

    lax.fori_loop(0, (NCHUNK + 7) // 8, body, 0)
    for b in range(8):
        pltpu.make_async_copy(ones_v, acc_sh.at[dst_v.at[0]], ssem[b]).wait()
    plsc.subcore_barrier()
    pltpu.sync_copy(acc_sh.at[pl.ds(s * DPT, DPT)], stage_v)
    pltpu.sync_copy(stage_v, out_hbm.at[c, s])


@functools.partial(
    pl.kernel,
    out_type=jax.ShapeDtypeStruct((NC, NS, RPT, H), jnp.float32),
    mesh=_mesh,
    compiler_params=_sc_params,
    scratch_types=[
        pltpu.VMEM((NCHUNK, CH), jnp.int32),
        pltpu.VMEM((NCHUNK, CH), jnp.int32),
        pltpu.VMEM((8, CH, H), jnp.float32),
        pltpu.VMEM((RPT, H), jnp.float32),
        pltpu.VMEM_SHARED((N, H), jnp.float32),
    ] + [pltpu.SemaphoreType.DMA] * 16,
)
def _edge_kernel(y_hbm, src_hbm, dst_hbm, out_hbm,
                 src_v, dst_v, rows, stage, acc_sh, *sems):
    gsem = sems[:8]
    ssem = sems[8:]
    c = lax.axis_index("c")
    s = lax.axis_index("s")

    def zero(i, _):
        stage[i, :] = jnp.zeros((H,), jnp.float32)
        return 0

    lax.fori_loop(0, RPT, zero, 0)
    pltpu.sync_copy(stage, acc_sh.at[pl.ds(s * RPT, RPT)])
    plsc.subcore_barrier()

    wid = s * NC + c
    pltpu.sync_copy(src_hbm.at[wid], src_v)
    pltpu.sync_copy(dst_hbm.at[wid], dst_v)

    def wait_g(b):
        pltpu.make_async_copy(y_hbm.at[src_v.at[0]], rows.at[b], gsem[b]).wait()

    def wait_s(b):
        pltpu.make_async_copy(
            rows.at[b], acc_sh.at[dst_v.at[0]], ssem[b]).wait()

    for b in range(6):
        pltpu.async_copy(y_hbm.at[src_v.at[b]], rows.at[b], gsem[b])

    def body(g, _):
        for b in range(8):
            j = g * 8 + b
            bf = (b + 6) % 8

            @pl.when(j + 6 < NCHUNK)
            def _():
                @pl.when(j >= 2)
                def _():
                    wait_s(bf)

                pltpu.async_copy(
                    y_hbm.at[src_v.at[j + 6]], rows.at[bf], gsem[bf])

            @pl.when(j < NCHUNK)
            def _():
                wait_g(b)
                pltpu.async_copy(
                    rows.at[b], acc_sh.at[dst_v.at[j]], ssem[b], add=True)

        return 0

    lax.fori_loop(0, (NCHUNK + 7) // 8, body, 0)
    for b in range(8):
        wait_s(b)

    plsc.subcore_barrier()
    pltpu.sync_copy(acc_sh.at[pl.ds(s * RPT, RPT)], stage)
    pltpu.sync_copy(stage, out_hbm.at[c, s])


def _prep_body(cnt_ref, x_ref, w1_ref, y1_ref, dis_ref):
    cnt = cnt_ref[0] + cnt_ref[1]
    deg = cnt[:N] + 1.0
    dis = lax.rsqrt(deg)
    xw = jnp.dot(x_ref[...], w1_ref[...], preferred_element_type=jnp.float32)
    y1_ref[...] = xw * dis[:, None]
    dis_ref[...] = dis


def _mid_body(acc_ref, y_ref, dis_ref, b_ref, w_ref, o_ref):
    dis = dis_ref[...][:, None]
    pre = (acc_ref[0] + acc_ref[1] + y_ref[...]) * dis + b_ref[...]
    h = jnp.maximum(pre, 0.0)
    o_ref[...] = jnp.dot(h, w_ref[...],
                         preferred_element_type=jnp.float32) * dis


def _head_body(acc_ref, y_ref, dis_ref, b3_ref, batch_ref,
               wm1_ref, bm1_ref, wm2_ref, bm2_ref, o_ref):
    dis = dis_ref[...][:, None]
    h3 = (acc_ref[0] + acc_ref[1] + y_ref[...]) * dis + b3_ref[...]
    gid = lax.broadcasted_iota(jnp.int32, (G, N), 0)
    onehot = (gid == batch_ref[...][None, :]).astype(jnp.float32)
    sums = jnp.dot(onehot, h3, preferred_element_type=jnp.float32)
    counts = jnp.sum(onehot, axis=1)
    pooled = sums / jnp.maximum(counts, 1.0)[:, None]
    z = jnp.maximum(
        jnp.dot(pooled, wm1_ref[...], preferred_element_type=jnp.float32)
        + bm1_ref[...], 0.0)
    z = jnp.dot(z, wm2_ref[...],
                preferred_element_type=jnp.float32) + bm2_ref[...]
    m = jnp.max(z, axis=-1, keepdims=True)
    e = z - m
    o_ref[...] = e - jnp.log(jnp.sum(jnp.exp(e), axis=-1, keepdims=True))


_prep = pl.pallas_call(
    _prep_body,
    out_shape=(jax.ShapeDtypeStruct((N, H), jnp.float32),
               jax.ShapeDtypeStruct((N,), jnp.float32)),
)

_mid = pl.pallas_call(
    _mid_body,
    out_shape=jax.ShapeDtypeStruct((N, H), jnp.float32),
)

_head = pl.pallas_call(
    _head_body,
    out_shape=jax.ShapeDtypeStruct((G, 10), jnp.float32),
)


def kernel(x, edge_index, batch, W1, b1, W2, b2, W3, b3, Wm1, bm1, Wm2, bm2):
    src3d = edge_index[0].astype(jnp.int32).reshape(NW, NCHUNK, CH)
    dst3d = edge_index[1].astype(jnp.int32).reshape(NW, NCHUNK, CH)
    cnt = _deg_kernel(dst3d).reshape(NC, DEG_PAD)
    y1, dis = _prep(cnt, x, W1)
    acc1 = _edge_kernel(y1, src3d, dst3d).reshape(NC, N, H)
    y2 = _mid(acc1, y1, dis, b1, W2)
    acc2 = _edge_kernel(y2, src3d, dst3d).reshape(NC, N, H)
    y3 = _mid(acc2, y2, dis, b2, W3)
    acc3 = _edge_kernel(y3, src3d, dst3d).reshape(NC, N, H)
    return _head(acc3, y3, dis, b3, batch.astype(jnp.int32), Wm1, bm1, Wm2, bm2)


# speedup vs baseline: 66.8922x; 1.0419x over previous
import functools

import jax
import jax.numpy as jnp
from jax import lax
from jax.experimental import pallas as pl
from jax.experimental.pallas import tpu as pltpu
from jax.experimental.pallas import tpu_sc as plsc

N = 10000
E = 320000
G = 64
D = 128
H = 16
NC = 2
NS = 16
NW = NC * NS
EPW = E // NW
CH = 80
NCHUNK = EPW // CH
RPT = N // NS
DEG_PAD = 10240
DPT = DEG_PAD // NS

_mesh = plsc.VectorSubcoreMesh(core_axis_name="c", subcore_axis_name="s")
_sc_params = pltpu.CompilerParams(use_tc_tiling_on_sc=False)


@functools.partial(
    pl.kernel,
    out_type=jax.ShapeDtypeStruct((NC, NS, DPT), jnp.float32),
    mesh=_mesh,
    compiler_params=_sc_params,
    scratch_types=[
        pltpu.VMEM((NCHUNK, CH), jnp.int32),
        pltpu.VMEM((CH,), jnp.float32),
        pltpu.VMEM((DPT,), jnp.float32),
        pltpu.VMEM_SHARED((DEG_PAD,), jnp.float32),
    ] + [pltpu.SemaphoreType.DMA] * 8,
)
def _deg_kernel(dst_hbm, out_hbm, dst_v, ones_v, stage_v, acc_sh, *ssem):
    c = lax.axis_index("c")
    s = lax.axis_index("s")

    def fill(i, _):
        ones_v[pl.ds(i * 16, 16)] = jnp.ones((16,), jnp.float32)
        return 0

    lax.fori_loop(0, CH // 16, fill, 0)
    ones = ones_v

    def zero(i, _):
        stage_v[pl.ds(i * 16, 16)] = jnp.zeros((16,), jnp.float32)
        return 0

    lax.fori_loop(0, DPT // 16, zero, 0)
    pltpu.sync_copy(stage_v, acc_sh.at[pl.ds(s * DPT, DPT)])
    plsc.subcore_barrier()

    wid = s * NC + c
    pltpu.sync_copy(dst_hbm.at[wid], dst_v)

    def body(g, _):
        for b in range(8):
            j = g * 8 + b

            @pl.when(j < NCHUNK)
            def _():
                @pl.when(j >= 8)
                def _():
                    pltpu.make_async_copy(
                        ones, acc_sh.at[dst_v.at[0]], ssem[b]).wait()

                pltpu.async_copy(
                    ones, acc_sh.at[dst_v.at[j]], ssem[b], add=True)

        return 0

    lax.fori_loop(0, (NCHUNK + 7) // 8, body, 0)
    for b in range(8):
        pltpu.make_async_copy(ones, acc_sh.at[dst_v.at[0]], ssem[b]).wait()
    plsc.subcore_barrier()
    pltpu.sync_copy(acc_sh.at[pl.ds(s * DPT, DPT)], stage_v)
    pltpu.sync_copy(stage_v, out_hbm.at[c, s])


@functools.partial(
    pl.kernel,
    out_type=jax.ShapeDtypeStruct((NC, NS, RPT, H), jnp.float32),
    mesh=_mesh,
    compiler_params=_sc_params,
    scratch_types=[
        pltpu.VMEM((NCHUNK, CH), jnp.int32),
        pltpu.VMEM((NCHUNK, CH), jnp.int32),
        pltpu.VMEM((12, CH, H), jnp.float32),
        pltpu.VMEM((RPT, H), jnp.float32),
        pltpu.VMEM_SHARED((N, H), jnp.float32),
    ] + [pltpu.SemaphoreType.DMA] * 26,
)
def _edge_kernel(y_hbm, src_hbm, dst_hbm, out_hbm,
                 src_v, dst_v, rows, stage, acc_sh, *sems):
    gsem = sems[:12]
    ssem = sems[12:24]
    isem = sems[24:]
    c = lax.axis_index("c")
    s = lax.axis_index("s")

    wid = s * NC + c
    pltpu.async_copy(src_hbm.at[wid], src_v, isem[0])
    pltpu.async_copy(dst_hbm.at[wid], dst_v, isem[1])

    def zero(i, _):
        stage[i, :] = jnp.zeros((H,), jnp.float32)
        return 0

    lax.fori_loop(0, RPT, zero, 0)
    pltpu.sync_copy(stage, acc_sh.at[pl.ds(s * RPT, RPT)])
    pltpu.make_async_copy(src_hbm.at[wid], src_v, isem[0]).wait()
    pltpu.make_async_copy(dst_hbm.at[wid], dst_v, isem[1]).wait()
    plsc.subcore_barrier()

    def wait_g(b):
        pltpu.make_async_copy(y_hbm.at[src_v.at[0]], rows.at[b], gsem[b]).wait()

    def wait_s(b):
        pltpu.make_async_copy(
            rows.at[b], acc_sh.at[dst_v.at[0]], ssem[b]).wait()

    for b in range(6):
        pltpu.async_copy(y_hbm.at[src_v.at[b]], rows.at[b], gsem[b])

    def body(g, _):
        for b in range(12):
            j = g * 12 + b
            bf = (b + 6) % 12

            @pl.when(j + 6 < NCHUNK)
            def _():
                @pl.when(j >= 6)
                def _():
                    wait_s(bf)

                pltpu.async_copy(
                    y_hbm.at[src_v.at[j + 6]], rows.at[bf], gsem[bf])

            @pl.when(j < NCHUNK)
            def _():
                wait_g(b)
                pltpu.async_copy(
                    rows.at[b], acc_sh.at[dst_v.at[j]], ssem[b], add=True)

        return 0

    lax.fori_loop(0, (NCHUNK + 11) // 12, body, 0)
    for b in range(12):
        wait_s(b)

    plsc.subcore_barrier()
    pltpu.sync_copy(acc_sh.at[pl.ds(s * RPT, RPT)], stage)
    pltpu.sync_copy(stage, out_hbm.at[c, s])


def _prep_body(cnt_ref, x_ref, w1_ref, y1_ref, dis_ref):
    cnt = cnt_ref[0] + cnt_ref[1]
    deg = cnt[:N] + 1.0
    dis = lax.rsqrt(deg)
    xw = jnp.dot(x_ref[...], w1_ref[...], preferred_element_type=jnp.float32)
    y1_ref[...] = xw * dis[:, None]
    dis_ref[...] = dis


def _mid_body(acc_ref, y_ref, dis_ref, b_ref, w_ref, o_ref):
    dis = dis_ref[...][:, None]
    pre = (acc_ref[0] + acc_ref[1] + y_ref[...]) * dis + b_ref[...]
    h = jnp.maximum(pre, 0.0)
    o_ref[...] = jnp.dot(h, w_ref[...],
                         preferred_element_type=jnp.float32) * dis


def _head_body(acc_ref, y_ref, dis_ref, b3_ref, batch_ref,
               wm1_ref, bm1_ref, wm2_ref, bm2_ref, o_ref):
    dis = dis_ref[...][:, None]
    h3 = (acc_ref[0] + acc_ref[1] + y_ref[...]) * dis + b3_ref[...]
    gid = lax.broadcasted_iota(jnp.int32, (G, N), 0)
    onehot = (gid == batch_ref[...][None, :]).astype(jnp.float32)
    sums = jnp.dot(onehot, h3, preferred_element_type=jnp.float32)
    counts = jnp.sum(onehot, axis=1)
    pooled = sums / jnp.maximum(counts, 1.0)[:, None]
    z = jnp.maximum(
        jnp.dot(pooled, wm1_ref[...], preferred_element_type=jnp.float32)
        + bm1_ref[...], 0.0)
    z = jnp.dot(z, wm2_ref[...],
                preferred_element_type=jnp.float32) + bm2_ref[...]
    m = jnp.max(z, axis=-1, keepdims=True)
    e = z - m
    o_ref[...] = e - jnp.log(jnp.sum(jnp.exp(e), axis=-1, keepdims=True))


_prep = pl.pallas_call(
    _prep_body,
    out_shape=(jax.ShapeDtypeStruct((N, H), jnp.float32),
               jax.ShapeDtypeStruct((N,), jnp.float32)),
)

_mid = pl.pallas_call(
    _mid_body,
    out_shape=jax.ShapeDtypeStruct((N, H), jnp.float32),
)

_head = pl.pallas_call(
    _head_body,
    out_shape=jax.ShapeDtypeStruct((G, 10), jnp.float32),
)


def kernel(x, edge_index, batch, W1, b1, W2, b2, W3, b3, Wm1, bm1, Wm2, bm2):
    src3d = edge_index[0].astype(jnp.int32).reshape(NW, NCHUNK, CH)
    dst3d = edge_index[1].astype(jnp.int32).reshape(NW, NCHUNK, CH)
    cnt = _deg_kernel(dst3d).reshape(NC, DEG_PAD)
    y1, dis = _prep(cnt, x, W1)
    acc1 = _edge_kernel(y1, src3d, dst3d).reshape(NC, N, H)
    y2 = _mid(acc1, y1, dis, b1, W2)
    acc2 = _edge_kernel(y2, src3d, dst3d).reshape(NC, N, H)
    y3 = _mid(acc2, y2, dis, b2, W3)
    acc3 = _edge_kernel(y3, src3d, dst3d).reshape(NC, N, H)
    return _head(acc3, y3, dis, b3, batch.astype(jnp.int32), Wm1, bm1, Wm2, bm2)
